# initial kernel scaffold (unmeasured)
import jax
import jax.numpy as jnp
from jax import lax
from jax.experimental import pallas as pl
from jax.experimental.pallas import tpu as pltpu

B, H, D, BS = 8, 8, 128, 16
NB = 512
PAGES = 512
CHUNK = 64
NCHUNK = PAGES // CHUNK
SCALE = D ** -0.5


def kernel(Q, K, V, bt, lens):
    def body(q_ref, k_hbm, v_hbm, bt_ref, lens_ref, out_ref,
             kbuf, vbuf, o_send, o_recv, st_send, st_recv,
             copy_sems, send_sems, recv_sems):
        my_x = lax.axis_index("x")
        my_y = lax.axis_index("y")
        my_z = lax.axis_index("z")
        peer = (1 - my_x, my_y, my_z)

        page_ids = my_x * PAGES + lax.broadcasted_iota(
            jnp.int32, (1, 1, PAGES), 2)
        j_ids = lax.broadcasted_iota(jnp.int32, (1, NB, 1), 1)
        lens_v = jnp.stack([lens_ref[i] for i in range(B)]).reshape(B, 1, 1)
        btv = bt_ref[:, :].reshape(B, NB, 1)
        eq = (btv == page_ids) & (j_ids < lens_v)
        counts = jnp.sum(eq.astype(jnp.float32), axis=1)

        q = q_ref[:, 0, :, :]

        def start_copy(c, slot):
            kc = pltpu.make_async_copy(
                k_hbm.at[pl.ds(c * CHUNK, CHUNK)], kbuf.at[slot],
                copy_sems.at[slot, 0])
            vc = pltpu.make_async_copy(
                v_hbm.at[pl.ds(c * CHUNK, CHUNK)], vbuf.at[slot],
                copy_sems.at[slot, 1])
            kc.start()
            vc.start()
            return kc, vc

        m = jnp.full((H, B), -1e30, jnp.float32)
        l = jnp.zeros((H, B), jnp.float32)
        acc = jnp.zeros((H, B, D), jnp.float32)

        pending = start_copy(0, 0)
        for c in range(NCHUNK):
            slot = c % 2
            if c + 1 < NCHUNK:
                nxt = start_copy(c + 1, (c + 1) % 2)
            pending[0].wait()
            pending[1].wait()
            kc = kbuf[slot].reshape(CHUNK * BS, H, D)
            vc = vbuf[slot].reshape(CHUNK * BS, H, D)
            s = lax.dot_general(
                q, kc, (((2,), (2,)), ((1,), (1,))),
                preferred_element_type=jnp.float32) * SCALE
            w = counts[:, c * CHUNK:(c + 1) * CHUNK]
            w = jnp.broadcast_to(
                w[:, :, None], (B, CHUNK, BS)).reshape(B, CHUNK * BS)
            m_new = jnp.maximum(m, jnp.max(s, axis=-1))
            p = jnp.exp(s - m_new[:, :, None]) * w[None]
            alpha = jnp.exp(m - m_new)
            l = l * alpha + jnp.sum(p, axis=-1)
            acc = acc * alpha[:, :, None] + lax.dot_general(
                p, vc, (((2,), (0,)), ((0,), (1,))),
                preferred_element_type=jnp.float32)
            m = m_new
            if c + 1 < NCHUNK:
                pending = nxt

        o_send[:, :, :] = acc
        st_send[0, :, :] = m
        st_send[1, :, :] = l

        bsem = pltpu.get_barrier_semaphore()
        pl.semaphore_signal(bsem, inc=1, device_id=peer,
                            device_id_type=pl.DeviceIdType.MESH)
        pl.semaphore_wait(bsem, 1)

        o_rdma = pltpu.make_async_remote_copy(
            src_ref=o_send, dst_ref=o_recv,
            send_sem=send_sems.at[0], recv_sem=recv_sems.at[0],
            device_id=peer, device_id_type=pl.DeviceIdType.MESH)
        st_rdma = pltpu.make_async_remote_copy(
            src_ref=st_send, dst_ref=st_recv,
            send_sem=send_sems.at[1], recv_sem=recv_sems.at[1],
            device_id=peer, device_id_type=pl.DeviceIdType.MESH)
        o_rdma.start()
        st_rdma.start()
        o_rdma.wait()
        st_rdma.wait()

        m_r = st_recv[0, :, :]
        l_r = st_recv[1, :, :]
        acc_r = o_recv[:, :, :]
        m_tot = jnp.maximum(m, m_r)
        a_l = jnp.exp(m - m_tot)
        a_r = jnp.exp(m_r - m_tot)
        denom = l * a_l + l_r * a_r
        o = (acc * a_l[:, :, None] + acc_r * a_r[:, :, None]) / denom[:, :, None]
        out_ref[:, 0, :, :] = jnp.transpose(o, (1, 0, 2))

    return pl.pallas_call(
        body,
        out_shape=jax.ShapeDtypeStruct((B, 1, H, D), jnp.float32),
        in_specs=[
            pl.BlockSpec(memory_space=pltpu.VMEM),
            pl.BlockSpec(memory_space=pltpu.ANY),
            pl.BlockSpec(memory_space=pltpu.ANY),
            pl.BlockSpec(memory_space=pltpu.VMEM),
            pl.BlockSpec(memory_space=pltpu.SMEM),
        ],
        out_specs=pl.BlockSpec(memory_space=pltpu.VMEM),
        scratch_shapes=[
            pltpu.VMEM((2, CHUNK, BS, H, D), jnp.float32),
            pltpu.VMEM((2, CHUNK, BS, H, D), jnp.float32),
            pltpu.VMEM((H, B, D), jnp.float32),
            pltpu.VMEM((H, B, D), jnp.float32),
            pltpu.VMEM((2, H, B), jnp.float32),
            pltpu.VMEM((2, H, B), jnp.float32),
            pltpu.SemaphoreType.DMA((2, 2)),
            pltpu.SemaphoreType.DMA((2,)),
            pltpu.SemaphoreType.DMA((2,)),
        ],
        compiler_params=pltpu.CompilerParams(collective_id=0),
    )(Q, K, V, bt, lens)


# baseline (device time: 55370 ns/iter reference)
import jax
import jax.numpy as jnp
from jax import lax
from jax.experimental import pallas as pl
from jax.experimental.pallas import tpu as pltpu

B, H, D, BS = 8, 8, 128, 16
NB = 512
PAGES = 512
CHUNK = 64
NCHUNK = PAGES // CHUNK
KT = CHUNK * BS
SCALE = D ** -0.5


def kernel(Q, K, V, bt, lens):
    def body(q_ref, k_hbm, v_hbm, bt_ref, lens_ref, out_ref,
             kbuf, vbuf, w_ref, o_send, o_recv, st_send, st_recv,
             copy_sems, send_sems, recv_sems):
        my_x = lax.axis_index("x")
        my_y = lax.axis_index("y")
        my_z = lax.axis_index("z")
        peer = (1 - my_x, my_y, my_z)

        j_ids = lax.broadcasted_iota(jnp.int32, (1, NB, 1), 1)
        lens_v = jnp.stack([lens_ref[i] for i in range(B)]).reshape(B, 1, 1)
        btv = bt_ref[:, :].reshape(B, NB, 1)
        valid = j_ids < lens_v
        for c in range(NCHUNK):
            page_ids = my_x * PAGES + c * CHUNK + lax.broadcasted_iota(
                jnp.int32, (1, 1, CHUNK), 2)
            eq = (btv == page_ids) & valid
            w_ref[c] = jnp.sum(eq.astype(jnp.float32), axis=1)

        def start_copy(c, slot):
            pltpu.make_async_copy(
                k_hbm.at[pl.ds(c * CHUNK, CHUNK)], kbuf.at[slot],
                copy_sems.at[slot, 0]).start()
            pltpu.make_async_copy(
                v_hbm.at[pl.ds(c * CHUNK, CHUNK)], vbuf.at[slot],
                copy_sems.at[slot, 1]).start()

        def wait_copy(c, slot):
            pltpu.make_async_copy(
                k_hbm.at[pl.ds(c * CHUNK, CHUNK)], kbuf.at[slot],
                copy_sems.at[slot, 0]).wait()
            pltpu.make_async_copy(
                v_hbm.at[pl.ds(c * CHUNK, CHUNK)], vbuf.at[slot],
                copy_sems.at[slot, 1]).wait()

        def step(c, slot, carry):
            ms, ls, accs = carry
            wait_copy(c, slot)
            wp = w_ref[c]
            w = jnp.broadcast_to(
                wp[:, :, None], (B, CHUNK, BS)).reshape(B, KT)
            new_ms, new_ls, new_accs = [], [], []
            for h in range(H):
                q_h = q_ref[:, 0, h, :]
                k_h = kbuf[slot, :, :, h, :].reshape(KT, D)
                v_h = vbuf[slot, :, :, h, :].reshape(KT, D)
                s = lax.dot_general(
                    q_h, k_h, (((1,), (1,)), ((), ())),
                    preferred_element_type=jnp.float32) * SCALE
                m_new = jnp.maximum(ms[h], jnp.max(s, 1, keepdims=True))
                p = jnp.exp(s - m_new) * w
                alpha = jnp.exp(ms[h] - m_new)
                new_ls.append(ls[h] * alpha + jnp.sum(p, 1, keepdims=True))
                new_accs.append(accs[h] * alpha + lax.dot_general(
                    p, v_h, (((1,), (0,)), ((), ())),
                    preferred_element_type=jnp.float32))
                new_ms.append(m_new)
            @pl.when(c + 2 < NCHUNK)
            def _():
                start_copy(c + 2, slot)
            return tuple(new_ms), tuple(new_ls), tuple(new_accs)

        start_copy(0, 0)
        start_copy(1, 1)

        def pair_body(c2, carry):
            c = c2 * 2
            carry = step(c, 0, carry)
            carry = step(c + 1, 1, carry)
            return carry

        m0 = tuple(jnp.full((B, 1), -1e30, jnp.float32) for _ in range(H))
        l0 = tuple(jnp.zeros((B, 1), jnp.float32) for _ in range(H))
        acc0 = tuple(jnp.zeros((B, D), jnp.float32) for _ in range(H))
        ms, ls, accs = lax.fori_loop(0, NCHUNK // 2, pair_body, (m0, l0, acc0))

        for h in range(H):
            o_send[h, :, :] = accs[h]
            st_send[0, :, h:h + 1] = ms[h]
            st_send[1, :, h:h + 1] = ls[h]

        bsem = pltpu.get_barrier_semaphore()
        pl.semaphore_signal(bsem, inc=1, device_id=peer,
                            device_id_type=pl.DeviceIdType.MESH)
        pl.semaphore_wait(bsem, 1)

        o_rdma = pltpu.make_async_remote_copy(
            src_ref=o_send, dst_ref=o_recv,
            send_sem=send_sems.at[0], recv_sem=recv_sems.at[0],
            device_id=peer, device_id_type=pl.DeviceIdType.MESH)
        st_rdma = pltpu.make_async_remote_copy(
            src_ref=st_send, dst_ref=st_recv,
            send_sem=send_sems.at[1], recv_sem=recv_sems.at[1],
            device_id=peer, device_id_type=pl.DeviceIdType.MESH)
        o_rdma.start()
        st_rdma.start()
        o_rdma.wait()
        st_rdma.wait()

        for h in range(H):
            m_r = st_recv[0, :, h:h + 1]
            l_r = st_recv[1, :, h:h + 1]
            acc_r = o_recv[h, :, :]
            m_tot = jnp.maximum(ms[h], m_r)
            a_l = jnp.exp(ms[h] - m_tot)
            a_r = jnp.exp(m_r - m_tot)
            denom = ls[h] * a_l + l_r * a_r
            out_ref[:, 0, h, :] = (accs[h] * a_l + acc_r * a_r) / denom

    return pl.pallas_call(
        body,
        out_shape=jax.ShapeDtypeStruct((B, 1, H, D), jnp.float32),
        in_specs=[
            pl.BlockSpec(memory_space=pltpu.VMEM),
            pl.BlockSpec(memory_space=pl.ANY),
            pl.BlockSpec(memory_space=pl.ANY),
            pl.BlockSpec(memory_space=pltpu.VMEM),
            pl.BlockSpec(memory_space=pltpu.SMEM),
        ],
        out_specs=pl.BlockSpec(memory_space=pltpu.VMEM),
        scratch_shapes=[
            pltpu.VMEM((2, CHUNK, BS, H, D), jnp.float32),
            pltpu.VMEM((2, CHUNK, BS, H, D), jnp.float32),
            pltpu.VMEM((NCHUNK, B, CHUNK), jnp.float32),
            pltpu.VMEM((H, B, D), jnp.float32),
            pltpu.VMEM((H, B, D), jnp.float32),
            pltpu.VMEM((2, B, H), jnp.float32),
            pltpu.VMEM((2, B, H), jnp.float32),
            pltpu.SemaphoreType.DMA((2, 2)),
            pltpu.SemaphoreType.DMA((2,)),
            pltpu.SemaphoreType.DMA((2,)),
        ],
        compiler_params=pltpu.CompilerParams(collective_id=0),
    )(Q, K, V, bt, lens)
